# baseline (speedup 1.0000x reference)
import functools

import jax
import jax.numpy as jnp
from jax import lax
from jax.experimental import pallas as pl
from jax.experimental.pallas import tpu as pltpu
from jax.experimental.pallas import tpu_sc as plsc

_TABLE_ROWS = 2048
_DIM = 2048
_BATCH = 16384

_info = plsc.get_sparse_core_info()
_NC = _info.num_cores
_NS = _info.num_subcores
_NW = _NC * _NS
_BPW = _BATCH // _NW
_CHUNK = 8
_NBUF = 4
_LOOK = 2
_NCHUNK = _BPW // _CHUNK
_NSUP = _NCHUNK // _NBUF

_mesh = plsc.VectorSubcoreMesh(core_axis_name="c", subcore_axis_name="s")


@functools.partial(
    pl.kernel,
    mesh=_mesh,
    out_type=jax.ShapeDtypeStruct((_BATCH, _DIM), jnp.float32),
    scratch_types=[
        pltpu.VMEM((_BPW,), jnp.int32),
        pltpu.VMEM((_NBUF, _CHUNK, _DIM), jnp.float32),
    ]
    + [pltpu.SemaphoreType.DMA] * (2 * _NBUF),
)
def _sc_gather(table_hbm, idx_hbm, out_hbm, idx_v, bufs, *sems):
    gsems = sems[:_NBUF]
    wsems = sems[_NBUF:]
    wid = lax.axis_index("s") * _NC + lax.axis_index("c")
    base = wid * _BPW
    pltpu.sync_copy(idx_hbm.at[pl.ds(base, _BPW)], idx_v)

    def gather(g, s):
        return pltpu.async_copy(
            table_hbm.at[idx_v.at[pl.ds(g * _CHUNK, _CHUNK)]], bufs.at[s],
            gsems[s])

    def wait_gather(g, s):
        pltpu.make_async_copy(
            table_hbm.at[idx_v.at[pl.ds(g * _CHUNK, _CHUNK)]], bufs.at[s],
            gsems[s]).wait()

    def scatter(g, s):
        return pltpu.async_copy(
            bufs.at[s], out_hbm.at[pl.ds(base + g * _CHUNK, _CHUNK)],
            wsems[s])

    def wait_scatter(g, s):
        pltpu.make_async_copy(
            bufs.at[s], out_hbm.at[pl.ds(base + g * _CHUNK, _CHUNK)],
            wsems[s]).wait()

    for c in range(_LOOK):
        gather(c, c % _NBUF)

    def wave(i, carry):
        for k in range(_NBUF):
            c = _NBUF * i + k
            s_ahead = (k + _LOOK) % _NBUF

            @pl.when(c + _LOOK < _NCHUNK)
            def _():
                @pl.when(c + _LOOK >= _NBUF)
                def _():
                    wait_scatter(c + _LOOK - _NBUF, s_ahead)

                gather(c + _LOOK, s_ahead)

            wait_gather(c, k)
            scatter(c, k)
        return carry

    lax.fori_loop(0, _NSUP, wave, 0)
    for k in range(_NBUF):
        wait_scatter(_NCHUNK - _NBUF + k, k)


def kernel(time, embeddings):
    idx = time.astype(jnp.int32)
    return _sc_gather(embeddings, idx)


# speedup vs baseline: 1.0008x; 1.0008x over previous
import functools

import jax
import jax.numpy as jnp
from jax import lax
from jax.experimental import pallas as pl
from jax.experimental.pallas import tpu as pltpu
from jax.experimental.pallas import tpu_sc as plsc

_TABLE_ROWS = 2048
_DIM = 2048
_BATCH = 16384

_info = plsc.get_sparse_core_info()
_NC = _info.num_cores
_NS = _info.num_subcores
_NW = _NC * _NS
_BPW = _BATCH // _NW
_CHUNK = 8
_NBUF = 4
_LOOK = 3
_NCHUNK = _BPW // _CHUNK
_NSUP = _NCHUNK // _NBUF

_mesh = plsc.VectorSubcoreMesh(core_axis_name="c", subcore_axis_name="s")


@functools.partial(
    pl.kernel,
    mesh=_mesh,
    out_type=jax.ShapeDtypeStruct((_BATCH, _DIM), jnp.float32),
    scratch_types=[
        pltpu.VMEM((_BPW,), jnp.int32),
        pltpu.VMEM((_NBUF, _CHUNK, _DIM), jnp.float32),
    ]
    + [pltpu.SemaphoreType.DMA] * (2 * _NBUF),
)
def _sc_gather(table_hbm, idx_hbm, out_hbm, idx_v, bufs, *sems):
    gsems = sems[:_NBUF]
    wsems = sems[_NBUF:]
    wid = lax.axis_index("s") * _NC + lax.axis_index("c")
    base = wid * _BPW
    pltpu.sync_copy(idx_hbm.at[pl.ds(base, _BPW)], idx_v)

    def gather(g, s):
        return pltpu.async_copy(
            table_hbm.at[idx_v.at[pl.ds(g * _CHUNK, _CHUNK)]], bufs.at[s],
            gsems[s])

    def wait_gather(g, s):
        pltpu.make_async_copy(
            table_hbm.at[idx_v.at[pl.ds(g * _CHUNK, _CHUNK)]], bufs.at[s],
            gsems[s]).wait()

    def scatter(g, s):
        return pltpu.async_copy(
            bufs.at[s], out_hbm.at[pl.ds(base + g * _CHUNK, _CHUNK)],
            wsems[s])

    def wait_scatter(g, s):
        pltpu.make_async_copy(
            bufs.at[s], out_hbm.at[pl.ds(base + g * _CHUNK, _CHUNK)],
            wsems[s]).wait()

    for c in range(_LOOK):
        gather(c, c % _NBUF)

    def wave(i, carry):
        for k in range(_NBUF):
            c = _NBUF * i + k
            s_ahead = (k + _LOOK) % _NBUF

            @pl.when(c + _LOOK < _NCHUNK)
            def _():
                @pl.when(c + _LOOK >= _NBUF)
                def _():
                    wait_scatter(c + _LOOK - _NBUF, s_ahead)

                gather(c + _LOOK, s_ahead)

            wait_gather(c, k)
            scatter(c, k)
        return carry

    lax.fori_loop(0, _NSUP, wave, 0)
    for k in range(_NBUF):
        wait_scatter(_NCHUNK - _NBUF + k, k)


def kernel(time, embeddings):
    idx = time.astype(jnp.int32)
    return _sc_gather(embeddings, idx)


# speedup vs baseline: 1.0048x; 1.0040x over previous
import functools

import jax
import jax.numpy as jnp
from jax import lax
from jax.experimental import pallas as pl
from jax.experimental.pallas import tpu as pltpu
from jax.experimental.pallas import tpu_sc as plsc

_TABLE_ROWS = 2048
_DIM = 2048
_BATCH = 16384

_info = plsc.get_sparse_core_info()
_NC = _info.num_cores
_NS = _info.num_subcores
_NW = _NC * _NS
_BPW = _BATCH // _NW
_CHUNK = 8
_NBUF = 4
_LOOK = 3
_NCHUNK = _BPW // _CHUNK
_NSUP = _NCHUNK // _NBUF

_mesh = plsc.VectorSubcoreMesh(core_axis_name="c", subcore_axis_name="s")


@functools.partial(
    pl.kernel,
    mesh=_mesh,
    out_type=jax.ShapeDtypeStruct((_BATCH, _DIM), jnp.float32),
    scratch_types=[
        pltpu.VMEM((_BPW,), jnp.int32),
        pltpu.VMEM((_NBUF, _CHUNK, _DIM), jnp.float32),
    ]
    + [pltpu.SemaphoreType.DMA] * (2 * _NBUF),
)
def _sc_gather(table_hbm, idx_hbm, out_hbm, idx_v, bufs, *sems):
    gsems = sems[:_NBUF]
    wsems = sems[_NBUF:]
    wid = lax.axis_index("s") * _NC + lax.axis_index("c")
    base = wid * _BPW
    phase = (wid % 2) * (_NCHUNK // 2)
    pltpu.sync_copy(idx_hbm.at[pl.ds(base, _BPW)], idx_v)

    def chunk_of(c):
        return lax.rem(c + phase, _NCHUNK)

    def gather(g, s):
        g = chunk_of(g)
        return pltpu.async_copy(
            table_hbm.at[idx_v.at[pl.ds(g * _CHUNK, _CHUNK)]], bufs.at[s],
            gsems[s])

    def wait_gather(g, s):
        g = chunk_of(g)
        pltpu.make_async_copy(
            table_hbm.at[idx_v.at[pl.ds(g * _CHUNK, _CHUNK)]], bufs.at[s],
            gsems[s]).wait()

    def scatter(g, s):
        g = chunk_of(g)
        return pltpu.async_copy(
            bufs.at[s], out_hbm.at[pl.ds(base + g * _CHUNK, _CHUNK)],
            wsems[s])

    def wait_scatter(g, s):
        g = chunk_of(g)
        pltpu.make_async_copy(
            bufs.at[s], out_hbm.at[pl.ds(base + g * _CHUNK, _CHUNK)],
            wsems[s]).wait()

    for c in range(_LOOK):
        gather(c, c % _NBUF)

    def wave(i, carry):
        for k in range(_NBUF):
            c = _NBUF * i + k
            s_ahead = (k + _LOOK) % _NBUF

            @pl.when(c + _LOOK < _NCHUNK)
            def _():
                @pl.when(c + _LOOK >= _NBUF)
                def _():
                    wait_scatter(c + _LOOK - _NBUF, s_ahead)

                gather(c + _LOOK, s_ahead)

            wait_gather(c, k)
            scatter(c, k)
        return carry

    lax.fori_loop(0, _NSUP, wave, 0)
    for k in range(_NBUF):
        wait_scatter(_NCHUNK - _NBUF + k, k)


def kernel(time, embeddings):
    idx = time.astype(jnp.int32)
    return _sc_gather(embeddings, idx)
